# Initial kernel scaffold; baseline (speedup 1.0000x reference)
import jax
import jax.numpy as jnp
from jax.experimental import pallas as pl


def kernel(x, edge_index, edge_attr, W_ne, b_ne, W_ee, b_ee, t, W1, b1, g1, be1, W2, b2, ln_g, ln_b, W_out, b_out):
    raise NotImplementedError("write your pallas kernel here")


# speedup vs baseline: 1.8226x; 1.8226x over previous
import functools

import jax
import jax.numpy as jnp
from jax import lax
from jax.experimental import pallas as pl

N_NODES = 10000
N_EDGES = 320000
HID = 128
HALF = 64
N_CLASSES = 112

ROW_BLK = 1000
EDGE_BLK = 2000


def _ln(z, g, b):
    mu = jnp.mean(z, axis=-1, keepdims=True)
    var = jnp.mean((z - mu) ** 2, axis=-1, keepdims=True)
    return (z - mu) / jnp.sqrt(var + 1e-5) * g + b


def _enc_body(x_ref, w_ref, b_ref, out_ref):
    z = jnp.dot(x_ref[...], w_ref[...], preferred_element_type=jnp.float32)
    z = z + b_ref[...]
    out_ref[0, :, :] = z[:, :HALF]
    out_ref[1, :, :] = z[:, HALF:]


def _encode_nodes(x, W, b):
    return pl.pallas_call(
        _enc_body,
        grid=(N_NODES // ROW_BLK,),
        in_specs=[
            pl.BlockSpec((ROW_BLK, 128), lambda i: (i, 0)),
            pl.BlockSpec((128, 128), lambda i: (0, 0)),
            pl.BlockSpec((1, 128), lambda i: (0, 0)),
        ],
        out_specs=pl.BlockSpec((2, ROW_BLK, HALF), lambda i: (0, i, 0)),
        out_shape=jax.ShapeDtypeStruct((2, N_NODES, HALF), jnp.float32),
    )(x, W, b.reshape(1, 128))


def _eenc_body(x_ref, w_ref, b_ref, out_ref):
    z = jnp.dot(x_ref[...], w_ref[...], preferred_element_type=jnp.float32)
    z = z + b_ref[...]
    out_ref[0, :, :] = z[:, :HALF]
    out_ref[1, :, :] = z[:, HALF:]


def _encode_edges(ea, W, b):
    return pl.pallas_call(
        _eenc_body,
        grid=(N_EDGES // EDGE_BLK,),
        in_specs=[
            pl.BlockSpec((EDGE_BLK, 16), lambda i: (i, 0)),
            pl.BlockSpec((16, 128), lambda i: (0, 0)),
            pl.BlockSpec((1, 128), lambda i: (0, 0)),
        ],
        out_specs=pl.BlockSpec((2, EDGE_BLK, HALF), lambda i: (0, i, 0)),
        out_shape=jax.ShapeDtypeStruct((2, N_EDGES, HALF), jnp.float32),
    )(ea, W, b.reshape(1, 128))


def _post_body(pex_ref, hs_ref, hp_ref, w1_ref, b1_ref, g1_ref, be1_ref,
               w2_ref, b2_ref, lng_ref, lnb_ref, hnew_ref, hsn_ref, *, first):
    num = jnp.concatenate([pex_ref[0, :, :HALF], pex_ref[1, :, :HALF]], axis=1)
    den = jnp.concatenate([pex_ref[0, :, HALF:], pex_ref[1, :, HALF:]], axis=1)
    hs = jnp.concatenate([hs_ref[0], hs_ref[1]], axis=1)
    out = num / (den + 1e-16) + hs
    z = jnp.dot(out, w1_ref[...], preferred_element_type=jnp.float32) + b1_ref[...]
    z = _ln(z, g1_ref[...], be1_ref[...])
    z = jnp.maximum(z, 0.0)
    z = jnp.dot(z, w2_ref[...], preferred_element_type=jnp.float32) + b2_ref[...]
    if first:
        hnew = z
    else:
        hnew = jnp.concatenate([hp_ref[0], hp_ref[1]], axis=1) + z
    hsn = jnp.maximum(_ln(hnew, lng_ref[...], lnb_ref[...]), 0.0)
    hnew_ref[0, :, :] = hnew[:, :HALF]
    hnew_ref[1, :, :] = hnew[:, HALF:]
    hsn_ref[0, :, :] = hsn[:, :HALF]
    hsn_ref[1, :, :] = hsn[:, HALF:]


def _layer_post(pex, hs, hprev, W1l, b1l, g1l, be1l, W2l, b2l, lngn, lnbn, first):
    split = pl.BlockSpec((2, ROW_BLK, HALF), lambda i: (0, i, 0))
    row1 = lambda n: pl.BlockSpec((1, n), lambda i: (0, 0))
    return pl.pallas_call(
        functools.partial(_post_body, first=first),
        grid=(N_NODES // ROW_BLK,),
        in_specs=[
            pl.BlockSpec((2, ROW_BLK, 128), lambda i: (0, i, 0)),
            split,
            split,
            pl.BlockSpec((128, 256), lambda i: (0, 0)),
            row1(256), row1(256), row1(256),
            pl.BlockSpec((256, 128), lambda i: (0, 0)),
            row1(128), row1(128), row1(128),
        ],
        out_specs=[split, split],
        out_shape=[
            jax.ShapeDtypeStruct((2, N_NODES, HALF), jnp.float32),
            jax.ShapeDtypeStruct((2, N_NODES, HALF), jnp.float32),
        ],
    )(pex, hs, hprev, W1l, b1l.reshape(1, 256), g1l.reshape(1, 256),
      be1l.reshape(1, 256), W2l, b2l.reshape(1, 128), lngn.reshape(1, 128),
      lnbn.reshape(1, 128))


def _final_body(hs_ref, w_ref, b_ref, out_ref):
    hs = jnp.concatenate([hs_ref[0], hs_ref[1]], axis=1)
    z = jnp.dot(hs, w_ref[...], preferred_element_type=jnp.float32) + b_ref[...]
    m = jnp.max(z, axis=-1, keepdims=True)
    lse = jnp.log(jnp.sum(jnp.exp(z - m), axis=-1, keepdims=True)) + m
    out_ref[...] = z - lse


def _final(hs, W, b):
    return pl.pallas_call(
        _final_body,
        grid=(N_NODES // ROW_BLK,),
        in_specs=[
            pl.BlockSpec((2, ROW_BLK, HALF), lambda i: (0, i, 0)),
            pl.BlockSpec((128, N_CLASSES), lambda i: (0, 0)),
            pl.BlockSpec((1, N_CLASSES), lambda i: (0, 0)),
        ],
        out_specs=pl.BlockSpec((ROW_BLK, N_CLASSES), lambda i: (i, 0)),
        out_shape=jax.ShapeDtypeStruct((N_NODES, N_CLASSES), jnp.float32),
    )(hs, W, b.reshape(1, N_CLASSES))


def _mp(src, dst, hs_split, ea_split, tl):
    hs = jnp.concatenate([hs_split[0], hs_split[1]], axis=1)
    ea = jnp.concatenate([ea_split[0], ea_split[1]], axis=1)
    msg = jnp.maximum(hs[src] + ea, 0.0) + 1e-7
    ex = jnp.exp(msg * tl)
    num = jax.ops.segment_sum(msg * ex, dst, num_segments=N_NODES)
    den = jax.ops.segment_sum(ex, dst, num_segments=N_NODES)
    pex = jnp.stack([
        jnp.concatenate([num[:, :HALF], den[:, :HALF]], axis=1),
        jnp.concatenate([num[:, HALF:], den[:, HALF:]], axis=1),
    ])
    return pex


def kernel(x, edge_index, edge_attr, W_ne, b_ne, W_ee, b_ee, t, W1, b1, g1,
           be1, W2, b2, ln_g, ln_b, W_out, b_out):
    src = edge_index[0]
    dst = edge_index[1]
    h_split = _encode_nodes(x, W_ne, b_ne)
    ea_split = _encode_edges(edge_attr, W_ee, b_ee)

    hs = h_split
    h = h_split
    for l in range(4):
        pex = _mp(src, dst, hs, ea_split, t[l])
        nxt = (l + 1) % 4
        h, hs = _layer_post(pex, hs, h, W1[l], b1[l], g1[l], be1[l], W2[l],
                            b2[l], ln_g[nxt], ln_b[nxt], first=(l == 0))
    return _final(hs, W_out, b_out)


# speedup vs baseline: 2.3662x; 1.2983x over previous
import functools

import jax
import jax.numpy as jnp
from jax import lax
from jax.experimental import pallas as pl
from jax.experimental.pallas import tpu as pltpu
from jax.experimental.pallas import tpu_sc as plsc

N_NODES = 10000
N_EDGES = 320000
HID = 128
HALF = 64
N_CLASSES = 112

ROW_BLK = 1000
EDGE_BLK = 2000


def _ln(z, g, b):
    mu = jnp.mean(z, axis=-1, keepdims=True)
    var = jnp.mean((z - mu) ** 2, axis=-1, keepdims=True)
    return (z - mu) / jnp.sqrt(var + 1e-5) * g + b


def _enc_body(x_ref, w_ref, b_ref, out_ref):
    z = jnp.dot(x_ref[...], w_ref[...], preferred_element_type=jnp.float32)
    z = z + b_ref[...]
    out_ref[0, :, :] = z[:, :HALF]
    out_ref[1, :, :] = z[:, HALF:]


def _encode_nodes(x, W, b):
    return pl.pallas_call(
        _enc_body,
        grid=(N_NODES // ROW_BLK,),
        in_specs=[
            pl.BlockSpec((ROW_BLK, 128), lambda i: (i, 0)),
            pl.BlockSpec((128, 128), lambda i: (0, 0)),
            pl.BlockSpec((1, 128), lambda i: (0, 0)),
        ],
        out_specs=pl.BlockSpec((2, ROW_BLK, HALF), lambda i: (0, i, 0)),
        out_shape=jax.ShapeDtypeStruct((2, N_NODES, HALF), jnp.float32),
    )(x, W, b.reshape(1, 128))


def _eenc_body(x_ref, w_ref, b_ref, out_ref):
    z = jnp.dot(x_ref[...], w_ref[...], preferred_element_type=jnp.float32)
    z = z + b_ref[...]
    out_ref[0, :, :] = z[:, :HALF]
    out_ref[1, :, :] = z[:, HALF:]


def _encode_edges(ea, W, b):
    return pl.pallas_call(
        _eenc_body,
        grid=(N_EDGES // EDGE_BLK,),
        in_specs=[
            pl.BlockSpec((EDGE_BLK, 16), lambda i: (i, 0)),
            pl.BlockSpec((16, 128), lambda i: (0, 0)),
            pl.BlockSpec((1, 128), lambda i: (0, 0)),
        ],
        out_specs=pl.BlockSpec((2, EDGE_BLK, HALF), lambda i: (0, i, 0)),
        out_shape=jax.ShapeDtypeStruct((2, N_EDGES, HALF), jnp.float32),
    )(ea, W, b.reshape(1, 128))


def _post_body(pex_ref, hs_ref, hp_ref, w1_ref, b1_ref, g1_ref, be1_ref,
               w2_ref, b2_ref, lng_ref, lnb_ref, hnew_ref, hsn_ref, *, first):
    num = jnp.concatenate([pex_ref[0, :, :HALF], pex_ref[1, :, :HALF]], axis=1)
    den = jnp.concatenate([pex_ref[0, :, HALF:], pex_ref[1, :, HALF:]], axis=1)
    hs = jnp.concatenate([hs_ref[0], hs_ref[1]], axis=1)
    out = num / (den + 1e-16) + hs
    z = jnp.dot(out, w1_ref[...], preferred_element_type=jnp.float32) + b1_ref[...]
    z = _ln(z, g1_ref[...], be1_ref[...])
    z = jnp.maximum(z, 0.0)
    z = jnp.dot(z, w2_ref[...], preferred_element_type=jnp.float32) + b2_ref[...]
    if first:
        hnew = z
    else:
        hnew = jnp.concatenate([hp_ref[0], hp_ref[1]], axis=1) + z
    hsn = jnp.maximum(_ln(hnew, lng_ref[...], lnb_ref[...]), 0.0)
    hnew_ref[0, :, :] = hnew[:, :HALF]
    hnew_ref[1, :, :] = hnew[:, HALF:]
    hsn_ref[0, :, :] = hsn[:, :HALF]
    hsn_ref[1, :, :] = hsn[:, HALF:]


def _layer_post(pex, hs, hprev, W1l, b1l, g1l, be1l, W2l, b2l, lngn, lnbn, first):
    split = pl.BlockSpec((2, ROW_BLK, HALF), lambda i: (0, i, 0))
    row1 = lambda n: pl.BlockSpec((1, n), lambda i: (0, 0))
    return pl.pallas_call(
        functools.partial(_post_body, first=first),
        grid=(N_NODES // ROW_BLK,),
        in_specs=[
            pl.BlockSpec((2, ROW_BLK, 128), lambda i: (0, i, 0)),
            split,
            split,
            pl.BlockSpec((128, 256), lambda i: (0, 0)),
            row1(256), row1(256), row1(256),
            pl.BlockSpec((256, 128), lambda i: (0, 0)),
            row1(128), row1(128), row1(128),
        ],
        out_specs=[split, split],
        out_shape=[
            jax.ShapeDtypeStruct((2, N_NODES, HALF), jnp.float32),
            jax.ShapeDtypeStruct((2, N_NODES, HALF), jnp.float32),
        ],
    )(pex, hs, hprev, W1l, b1l.reshape(1, 256), g1l.reshape(1, 256),
      be1l.reshape(1, 256), W2l, b2l.reshape(1, 128), lngn.reshape(1, 128),
      lnbn.reshape(1, 128))


def _final_body(hs_ref, w_ref, b_ref, out_ref):
    hs = jnp.concatenate([hs_ref[0], hs_ref[1]], axis=1)
    z = jnp.dot(hs, w_ref[...], preferred_element_type=jnp.float32) + b_ref[...]
    m = jnp.max(z, axis=-1, keepdims=True)
    lse = jnp.log(jnp.sum(jnp.exp(z - m), axis=-1, keepdims=True)) + m
    out_ref[...] = z - lse


def _final(hs, W, b):
    return pl.pallas_call(
        _final_body,
        grid=(N_NODES // ROW_BLK,),
        in_specs=[
            pl.BlockSpec((2, ROW_BLK, HALF), lambda i: (0, i, 0)),
            pl.BlockSpec((128, N_CLASSES), lambda i: (0, 0)),
            pl.BlockSpec((1, N_CLASSES), lambda i: (0, 0)),
        ],
        out_specs=pl.BlockSpec((ROW_BLK, N_CLASSES), lambda i: (i, 0)),
        out_shape=jax.ShapeDtypeStruct((N_NODES, N_CLASSES), jnp.float32),
    )(hs, W, b.reshape(1, N_CLASSES))


MP_BLK = 128
MP_NBLK = N_EDGES // MP_BLK
NS = 16
NC = 2
ZCH = N_NODES // MP_BLK
ZTAIL = N_NODES - ZCH * MP_BLK


def _mp_body(src_h, dst_h, hs_h, ea_h, t_h, out_h,
             acc_sh, sidx_v, sidx2_v, didx_v, hrows_v, ea_v, pex_v, tvec_v,
             sem):
    c = lax.axis_index("c")
    s = lax.axis_index("s")

    pltpu.sync_copy(t_h, tvec_v)
    tv = tvec_v[...]

    def _zrow(e, _):
        for j in range(8):
            pex_v[e, pl.ds(j * 16, 16)] = jnp.zeros((16,), jnp.float32)
        return _
    lax.fori_loop(0, MP_BLK, _zrow, 0)

    nz = (ZCH - s + NS - 1) // NS
    def _zchunk(i, _):
        k = s + i * NS
        pltpu.sync_copy(pex_v, acc_sh.at[pl.ds(k * MP_BLK, MP_BLK), :])
        return _
    lax.fori_loop(0, nz, _zchunk, 0)

    @pl.when(s == ZCH % NS)
    def _ztail():
        pltpu.sync_copy(pex_v.at[pl.ds(0, ZTAIL), :],
                        acc_sh.at[pl.ds(ZCH * MP_BLK, ZTAIL), :])

    plsc.subcore_barrier()

    nb = (MP_NBLK - s + NS - 1) // NS

    def _block(i, _):
        base = (s + i * NS) * MP_BLK
        pltpu.sync_copy(src_h.at[pl.ds(base, MP_BLK)], sidx_v)
        pltpu.sync_copy(dst_h.at[pl.ds(base, MP_BLK)], didx_v)
        off = (c * N_NODES).astype(jnp.int32)
        for j in range(MP_BLK // 16):
            sidx2_v[pl.ds(j * 16, 16)] = sidx_v[pl.ds(j * 16, 16)] + off
        pltpu.async_copy(hs_h.at[sidx2_v], hrows_v, sem).wait()
        pltpu.sync_copy(ea_h.at[pl.ds(c * N_EDGES + base, MP_BLK), :], ea_v)

        def _erow(e, _):
            for j in range(4):
                hv = hrows_v[e, pl.ds(j * 16, 16)]
                av = ea_v[e, pl.ds(j * 16, 16)]
                m = jnp.maximum(hv + av, 0.0) + 1e-7
                ex = jnp.exp(m * tv)
                pex_v[e, pl.ds(j * 16, 16)] = m * ex
                pex_v[e, pl.ds(64 + j * 16, 16)] = ex
            return _
        lax.fori_loop(0, MP_BLK, _erow, 0)

        pltpu.sync_copy(pex_v, acc_sh.at[didx_v], add=True)
        return _

    lax.fori_loop(0, nb, _block, 0)

    plsc.subcore_barrier()

    def _wchunk(i, _):
        k = s + i * NS
        pltpu.sync_copy(acc_sh.at[pl.ds(k * MP_BLK, MP_BLK), :],
                        out_h.at[pl.ds(c * N_NODES + k * MP_BLK, MP_BLK), :])
        return _
    lax.fori_loop(0, nz, _wchunk, 0)

    @pl.when(s == ZCH % NS)
    def _wtail():
        pltpu.sync_copy(acc_sh.at[pl.ds(ZCH * MP_BLK, ZTAIL), :],
                        out_h.at[pl.ds(c * N_NODES + ZCH * MP_BLK, ZTAIL), :])


@functools.partial(jax.jit, static_argnames=())
def _mp_sc(src, dst, hs2, ea2, t_arr):
    mesh = plsc.VectorSubcoreMesh(core_axis_name="c", subcore_axis_name="s",
                                  num_cores=NC, num_subcores=NS)
    f = pl.kernel(
        _mp_body,
        out_type=jax.ShapeDtypeStruct((NC * N_NODES, 128), jnp.float32),
        mesh=mesh,
        compiler_params=pltpu.CompilerParams(use_tc_tiling_on_sc=False),
        scratch_types=[
            pltpu.VMEM_SHARED((N_NODES, 128), jnp.float32),
            pltpu.VMEM((MP_BLK,), jnp.int32),
            pltpu.VMEM((MP_BLK,), jnp.int32),
            pltpu.VMEM((MP_BLK,), jnp.int32),
            pltpu.VMEM((MP_BLK, HALF), jnp.float32),
            pltpu.VMEM((MP_BLK, HALF), jnp.float32),
            pltpu.VMEM((MP_BLK, 128), jnp.float32),
            pltpu.VMEM((16,), jnp.float32),
            pltpu.SemaphoreType.DMA,
        ],
    )
    return f(src, dst, hs2, ea2, t_arr)


def _mp(src, dst, hs_split, ea2, tl):
    hs2 = hs_split.reshape(NC * N_NODES, HALF)
    t_arr = jnp.full((16,), tl, jnp.float32)
    pex = _mp_sc(src, dst, hs2, ea2, t_arr)
    return pex.reshape(NC, N_NODES, 128)


def kernel(x, edge_index, edge_attr, W_ne, b_ne, W_ee, b_ee, t, W1, b1, g1,
           be1, W2, b2, ln_g, ln_b, W_out, b_out):
    src = edge_index[0]
    dst = edge_index[1]
    h_split = _encode_nodes(x, W_ne, b_ne)
    ea_split = _encode_edges(edge_attr, W_ee, b_ee)
    ea2 = ea_split.reshape(NC * N_EDGES, HALF)

    hs = h_split
    h = h_split
    for l in range(4):
        pex = _mp(src, dst, hs, ea2, t[l])
        nxt = (l + 1) % 4
        h, hs = _layer_post(pex, hs, h, W1[l], b1[l], g1[l], be1[l], W2[l],
                            b2[l], ln_g[nxt], ln_b[nxt], first=(l == 0))
    return _final(hs, W_out, b_out)


# speedup vs baseline: 5.5612x; 2.3502x over previous
import functools

import jax
import jax.numpy as jnp
from jax import lax
from jax.experimental import pallas as pl
from jax.experimental.pallas import tpu as pltpu
from jax.experimental.pallas import tpu_sc as plsc

N_NODES = 10000
N_EDGES = 320000
HID = 128
HALF = 64
N_CLASSES = 112

ROW_BLK = 1000
EDGE_BLK = 2000


def _ln(z, g, b):
    mu = jnp.mean(z, axis=-1, keepdims=True)
    var = jnp.mean((z - mu) ** 2, axis=-1, keepdims=True)
    return (z - mu) / jnp.sqrt(var + 1e-5) * g + b


def _enc_body(x_ref, w_ref, b_ref, out_ref):
    z = jnp.dot(x_ref[...], w_ref[...], preferred_element_type=jnp.float32)
    z = z + b_ref[...]
    out_ref[0, :, :] = z[:, :HALF]
    out_ref[1, :, :] = z[:, HALF:]


def _encode_nodes(x, W, b):
    return pl.pallas_call(
        _enc_body,
        grid=(N_NODES // ROW_BLK,),
        in_specs=[
            pl.BlockSpec((ROW_BLK, 128), lambda i: (i, 0)),
            pl.BlockSpec((128, 128), lambda i: (0, 0)),
            pl.BlockSpec((1, 128), lambda i: (0, 0)),
        ],
        out_specs=pl.BlockSpec((2, ROW_BLK, HALF), lambda i: (0, i, 0)),
        out_shape=jax.ShapeDtypeStruct((2, N_NODES, HALF), jnp.float32),
    )(x, W, b.reshape(1, 128))


def _eenc_body(x_ref, w_ref, b_ref, out_ref):
    z = jnp.dot(x_ref[...], w_ref[...], preferred_element_type=jnp.float32)
    z = z + b_ref[...]
    out_ref[0, :, :] = z[:, :HALF]
    out_ref[1, :, :] = z[:, HALF:]


def _encode_edges(ea, W, b):
    return pl.pallas_call(
        _eenc_body,
        grid=(N_EDGES // EDGE_BLK,),
        in_specs=[
            pl.BlockSpec((EDGE_BLK, 16), lambda i: (i, 0)),
            pl.BlockSpec((16, 128), lambda i: (0, 0)),
            pl.BlockSpec((1, 128), lambda i: (0, 0)),
        ],
        out_specs=pl.BlockSpec((2, EDGE_BLK, HALF), lambda i: (0, i, 0)),
        out_shape=jax.ShapeDtypeStruct((2, N_EDGES, HALF), jnp.float32),
    )(ea, W, b.reshape(1, 128))


def _post_body(pex_ref, hs_ref, hp_ref, w1_ref, b1_ref, g1_ref, be1_ref,
               w2_ref, b2_ref, lng_ref, lnb_ref, hnew_ref, hsn_ref, *, first):
    num = jnp.concatenate([pex_ref[0, :, :HALF], pex_ref[1, :, :HALF]], axis=1)
    den = jnp.concatenate([pex_ref[0, :, HALF:], pex_ref[1, :, HALF:]], axis=1)
    hs = jnp.concatenate([hs_ref[0], hs_ref[1]], axis=1)
    out = num / (den + 1e-16) + hs
    z = jnp.dot(out, w1_ref[...], preferred_element_type=jnp.float32) + b1_ref[...]
    z = _ln(z, g1_ref[...], be1_ref[...])
    z = jnp.maximum(z, 0.0)
    z = jnp.dot(z, w2_ref[...], preferred_element_type=jnp.float32) + b2_ref[...]
    if first:
        hnew = z
    else:
        hnew = jnp.concatenate([hp_ref[0], hp_ref[1]], axis=1) + z
    hsn = jnp.maximum(_ln(hnew, lng_ref[...], lnb_ref[...]), 0.0)
    hnew_ref[0, :, :] = hnew[:, :HALF]
    hnew_ref[1, :, :] = hnew[:, HALF:]
    hsn_ref[0, :, :] = hsn[:, :HALF]
    hsn_ref[1, :, :] = hsn[:, HALF:]


def _layer_post(pex, hs, hprev, W1l, b1l, g1l, be1l, W2l, b2l, lngn, lnbn, first):
    split = pl.BlockSpec((2, ROW_BLK, HALF), lambda i: (0, i, 0))
    row1 = lambda n: pl.BlockSpec((1, n), lambda i: (0, 0))
    return pl.pallas_call(
        functools.partial(_post_body, first=first),
        grid=(N_NODES // ROW_BLK,),
        in_specs=[
            pl.BlockSpec((2, ROW_BLK, 128), lambda i: (0, i, 0)),
            split,
            split,
            pl.BlockSpec((128, 256), lambda i: (0, 0)),
            row1(256), row1(256), row1(256),
            pl.BlockSpec((256, 128), lambda i: (0, 0)),
            row1(128), row1(128), row1(128),
        ],
        out_specs=[split, split],
        out_shape=[
            jax.ShapeDtypeStruct((2, N_NODES, HALF), jnp.float32),
            jax.ShapeDtypeStruct((2, N_NODES, HALF), jnp.float32),
        ],
    )(pex, hs, hprev, W1l, b1l.reshape(1, 256), g1l.reshape(1, 256),
      be1l.reshape(1, 256), W2l, b2l.reshape(1, 128), lngn.reshape(1, 128),
      lnbn.reshape(1, 128))


def _final_body(hs_ref, w_ref, b_ref, out_ref):
    hs = jnp.concatenate([hs_ref[0], hs_ref[1]], axis=1)
    z = jnp.dot(hs, w_ref[...], preferred_element_type=jnp.float32) + b_ref[...]
    m = jnp.max(z, axis=-1, keepdims=True)
    lse = jnp.log(jnp.sum(jnp.exp(z - m), axis=-1, keepdims=True)) + m
    out_ref[...] = z - lse


def _final(hs, W, b):
    return pl.pallas_call(
        _final_body,
        grid=(N_NODES // ROW_BLK,),
        in_specs=[
            pl.BlockSpec((2, ROW_BLK, HALF), lambda i: (0, i, 0)),
            pl.BlockSpec((128, N_CLASSES), lambda i: (0, 0)),
            pl.BlockSpec((1, N_CLASSES), lambda i: (0, 0)),
        ],
        out_specs=pl.BlockSpec((ROW_BLK, N_CLASSES), lambda i: (i, 0)),
        out_shape=jax.ShapeDtypeStruct((N_NODES, N_CLASSES), jnp.float32),
    )(hs, W, b.reshape(1, N_CLASSES))


MP_BLK = 128
MP_NBLK = N_EDGES // MP_BLK
NS = 16
NC = 2
ZCH = N_NODES // MP_BLK
ZTAIL = N_NODES - ZCH * MP_BLK


def _mp_body(src_h, dst_h, hs_h, ea_h, t_h, out_h,
             acc_sh, sidx_v, sidx2_v, didx_v, hrows_v, ea_v, pex_v, tvec_v,
             sem):
    c = lax.axis_index("c")
    s = lax.axis_index("s")

    pltpu.sync_copy(t_h, tvec_v)
    tv = tvec_v[...]

    def _zrow(e, _):
        for j in range(8):
            pex_v[e, pl.ds(j * 16, 16)] = jnp.zeros((16,), jnp.float32)
        return _
    lax.fori_loop(0, MP_BLK, _zrow, 0)

    nz = (ZCH - s + NS - 1) // NS
    def _zchunk(i, _):
        k = s + i * NS
        pltpu.sync_copy(pex_v, acc_sh.at[pl.ds(k * MP_BLK, MP_BLK), :])
        return _
    lax.fori_loop(0, nz, _zchunk, 0)

    @pl.when(s == ZCH % NS)
    def _ztail():
        pltpu.sync_copy(pex_v.at[pl.ds(0, ZTAIL), :],
                        acc_sh.at[pl.ds(ZCH * MP_BLK, ZTAIL), :])

    plsc.subcore_barrier()

    nb = (MP_NBLK - s + NS - 1) // NS

    def _block(i, _):
        base = (s + i * NS) * MP_BLK
        pltpu.sync_copy(src_h.at[pl.ds(base, MP_BLK)], sidx_v)
        pltpu.sync_copy(dst_h.at[pl.ds(base, MP_BLK)], didx_v)
        off = (c * N_NODES).astype(jnp.int32)
        for j in range(MP_BLK // 16):
            sidx2_v[pl.ds(j * 16, 16)] = sidx_v[pl.ds(j * 16, 16)] + off
        pltpu.async_copy(hs_h.at[sidx2_v], hrows_v, sem).wait()
        pltpu.sync_copy(ea_h.at[pl.ds(c * N_EDGES + base, MP_BLK), :], ea_v)

        @plsc.parallel_loop(0, MP_BLK, unroll=4)
        def _erow(e):
            for j in range(4):
                hv = hrows_v[e, pl.ds(j * 16, 16)]
                av = ea_v[e, pl.ds(j * 16, 16)]
                m = jnp.maximum(hv + av, 0.0) + 1e-7
                ex = jnp.exp(m * tv)
                pex_v[e, pl.ds(j * 16, 16)] = m * ex
                pex_v[e, pl.ds(64 + j * 16, 16)] = ex

        pltpu.sync_copy(pex_v, acc_sh.at[didx_v], add=True)
        return _

    lax.fori_loop(0, nb, _block, 0)

    plsc.subcore_barrier()

    def _wchunk(i, _):
        k = s + i * NS
        pltpu.sync_copy(acc_sh.at[pl.ds(k * MP_BLK, MP_BLK), :],
                        out_h.at[pl.ds(c * N_NODES + k * MP_BLK, MP_BLK), :])
        return _
    lax.fori_loop(0, nz, _wchunk, 0)

    @pl.when(s == ZCH % NS)
    def _wtail():
        pltpu.sync_copy(acc_sh.at[pl.ds(ZCH * MP_BLK, ZTAIL), :],
                        out_h.at[pl.ds(c * N_NODES + ZCH * MP_BLK, ZTAIL), :])


@functools.partial(jax.jit, static_argnames=())
def _mp_sc(src, dst, hs2, ea2, t_arr):
    mesh = plsc.VectorSubcoreMesh(core_axis_name="c", subcore_axis_name="s",
                                  num_cores=NC, num_subcores=NS)
    f = pl.kernel(
        _mp_body,
        out_type=jax.ShapeDtypeStruct((NC * N_NODES, 128), jnp.float32),
        mesh=mesh,
        compiler_params=pltpu.CompilerParams(use_tc_tiling_on_sc=False),
        scratch_types=[
            pltpu.VMEM_SHARED((N_NODES, 128), jnp.float32),
            pltpu.VMEM((MP_BLK,), jnp.int32),
            pltpu.VMEM((MP_BLK,), jnp.int32),
            pltpu.VMEM((MP_BLK,), jnp.int32),
            pltpu.VMEM((MP_BLK, HALF), jnp.float32),
            pltpu.VMEM((MP_BLK, HALF), jnp.float32),
            pltpu.VMEM((MP_BLK, 128), jnp.float32),
            pltpu.VMEM((16,), jnp.float32),
            pltpu.SemaphoreType.DMA,
        ],
    )
    return f(src, dst, hs2, ea2, t_arr)


def _mp(src, dst, hs_split, ea2, tl):
    hs2 = hs_split.reshape(NC * N_NODES, HALF)
    t_arr = jnp.full((16,), tl, jnp.float32)
    pex = _mp_sc(src, dst, hs2, ea2, t_arr)
    return pex.reshape(NC, N_NODES, 128)


def kernel(x, edge_index, edge_attr, W_ne, b_ne, W_ee, b_ee, t, W1, b1, g1,
           be1, W2, b2, ln_g, ln_b, W_out, b_out):
    src = edge_index[0]
    dst = edge_index[1]
    h_split = _encode_nodes(x, W_ne, b_ne)
    ea_split = _encode_edges(edge_attr, W_ee, b_ee)
    ea2 = ea_split.reshape(NC * N_EDGES, HALF)

    hs = h_split
    h = h_split
    for l in range(4):
        pex = _mp(src, dst, hs, ea2, t[l])
        nxt = (l + 1) % 4
        h, hs = _layer_post(pex, hs, h, W1[l], b1[l], g1[l], be1[l], W2[l],
                            b2[l], ln_g[nxt], ln_b[nxt], first=(l == 0))
    return _final(hs, W_out, b_out)


# speedup vs baseline: 9.0363x; 1.6249x over previous
import functools

import jax
import jax.numpy as jnp
from jax import lax
from jax.experimental import pallas as pl
from jax.experimental.pallas import tpu as pltpu
from jax.experimental.pallas import tpu_sc as plsc

N_NODES = 10000
N_EDGES = 320000
HID = 128
HALF = 64
N_CLASSES = 112

ROW_BLK = 1000
EDGE_BLK = 2000


def _ln(z, g, b):
    mu = jnp.mean(z, axis=-1, keepdims=True)
    var = jnp.mean((z - mu) ** 2, axis=-1, keepdims=True)
    return (z - mu) / jnp.sqrt(var + 1e-5) * g + b


def _enc_body(x_ref, w_ref, b_ref, out_ref):
    z = jnp.dot(x_ref[...], w_ref[...], preferred_element_type=jnp.float32)
    z = z + b_ref[...]
    out_ref[0, :, :] = z[:, :HALF]
    out_ref[1, :, :] = z[:, HALF:]


def _encode_nodes(x, W, b):
    return pl.pallas_call(
        _enc_body,
        grid=(N_NODES // ROW_BLK,),
        in_specs=[
            pl.BlockSpec((ROW_BLK, 128), lambda i: (i, 0)),
            pl.BlockSpec((128, 128), lambda i: (0, 0)),
            pl.BlockSpec((1, 128), lambda i: (0, 0)),
        ],
        out_specs=pl.BlockSpec((2, ROW_BLK, HALF), lambda i: (0, i, 0)),
        out_shape=jax.ShapeDtypeStruct((2, N_NODES, HALF), jnp.float32),
    )(x, W, b.reshape(1, 128))


def _eenc_body(x_ref, w_ref, b_ref, out_ref):
    z = jnp.dot(x_ref[...], w_ref[...], preferred_element_type=jnp.float32)
    z = z + b_ref[...]
    out_ref[0, :, :] = z[:, :HALF]
    out_ref[1, :, :] = z[:, HALF:]


def _encode_edges(ea, W, b):
    return pl.pallas_call(
        _eenc_body,
        grid=(N_EDGES // EDGE_BLK,),
        in_specs=[
            pl.BlockSpec((EDGE_BLK, 16), lambda i: (i, 0)),
            pl.BlockSpec((16, 128), lambda i: (0, 0)),
            pl.BlockSpec((1, 128), lambda i: (0, 0)),
        ],
        out_specs=pl.BlockSpec((2, EDGE_BLK, HALF), lambda i: (0, i, 0)),
        out_shape=jax.ShapeDtypeStruct((2, N_EDGES, HALF), jnp.float32),
    )(ea, W, b.reshape(1, 128))


def _post_body(pex_ref, hs_ref, hp_ref, w1_ref, b1_ref, g1_ref, be1_ref,
               w2_ref, b2_ref, lng_ref, lnb_ref, hnew_ref, hsn_ref, *, first):
    num = jnp.concatenate([pex_ref[0, :, :HALF], pex_ref[1, :, :HALF]], axis=1)
    den = jnp.concatenate([pex_ref[0, :, HALF:], pex_ref[1, :, HALF:]], axis=1)
    hs = jnp.concatenate([hs_ref[0], hs_ref[1]], axis=1)
    out = num / (den + 1e-16) + hs
    z = jnp.dot(out, w1_ref[...], preferred_element_type=jnp.float32) + b1_ref[...]
    z = _ln(z, g1_ref[...], be1_ref[...])
    z = jnp.maximum(z, 0.0)
    z = jnp.dot(z, w2_ref[...], preferred_element_type=jnp.float32) + b2_ref[...]
    if first:
        hnew = z
    else:
        hnew = jnp.concatenate([hp_ref[0], hp_ref[1]], axis=1) + z
    hsn = jnp.maximum(_ln(hnew, lng_ref[...], lnb_ref[...]), 0.0)
    hnew_ref[0, :, :] = hnew[:, :HALF]
    hnew_ref[1, :, :] = hnew[:, HALF:]
    hsn_ref[0, :, :] = hsn[:, :HALF]
    hsn_ref[1, :, :] = hsn[:, HALF:]


def _layer_post(pex, hs, hprev, W1l, b1l, g1l, be1l, W2l, b2l, lngn, lnbn, first):
    split = pl.BlockSpec((2, ROW_BLK, HALF), lambda i: (0, i, 0))
    row1 = lambda n: pl.BlockSpec((1, n), lambda i: (0, 0))
    return pl.pallas_call(
        functools.partial(_post_body, first=first),
        grid=(N_NODES // ROW_BLK,),
        in_specs=[
            pl.BlockSpec((2, ROW_BLK, 128), lambda i: (0, i, 0)),
            split,
            split,
            pl.BlockSpec((128, 256), lambda i: (0, 0)),
            row1(256), row1(256), row1(256),
            pl.BlockSpec((256, 128), lambda i: (0, 0)),
            row1(128), row1(128), row1(128),
        ],
        out_specs=[split, split],
        out_shape=[
            jax.ShapeDtypeStruct((2, N_NODES, HALF), jnp.float32),
            jax.ShapeDtypeStruct((2, N_NODES, HALF), jnp.float32),
        ],
    )(pex, hs, hprev, W1l, b1l.reshape(1, 256), g1l.reshape(1, 256),
      be1l.reshape(1, 256), W2l, b2l.reshape(1, 128), lngn.reshape(1, 128),
      lnbn.reshape(1, 128))


def _final_body(hs_ref, w_ref, b_ref, out_ref):
    hs = jnp.concatenate([hs_ref[0], hs_ref[1]], axis=1)
    z = jnp.dot(hs, w_ref[...], preferred_element_type=jnp.float32) + b_ref[...]
    m = jnp.max(z, axis=-1, keepdims=True)
    lse = jnp.log(jnp.sum(jnp.exp(z - m), axis=-1, keepdims=True)) + m
    out_ref[...] = z - lse


def _final(hs, W, b):
    return pl.pallas_call(
        _final_body,
        grid=(N_NODES // ROW_BLK,),
        in_specs=[
            pl.BlockSpec((2, ROW_BLK, HALF), lambda i: (0, i, 0)),
            pl.BlockSpec((128, N_CLASSES), lambda i: (0, 0)),
            pl.BlockSpec((1, N_CLASSES), lambda i: (0, 0)),
        ],
        out_specs=pl.BlockSpec((ROW_BLK, N_CLASSES), lambda i: (i, 0)),
        out_shape=jax.ShapeDtypeStruct((N_NODES, N_CLASSES), jnp.float32),
    )(hs, W, b.reshape(1, N_CLASSES))


MP_BLK = 64
MP_NBLK = N_EDGES // MP_BLK
NS = 16
NC = 2
ZCH = N_NODES // MP_BLK
ZTAIL = N_NODES - ZCH * MP_BLK
NT_BASE = MP_NBLK // NS
NT_REM = MP_NBLK - NT_BASE * NS


def _mp_body(src_h, dst_h, hs_h, ea_h, t_h, out_h,
             acc_sh,
             si0, si1, si2, si3, di0, di1, di2, di3,
             hrows0, hrows1, ea0, ea1, pex0, pex1, tvec_v,
             ssi0, ssi1, ssi2, ssi3, sdi0, sdi1, sdi2, sdi3,
             sg0, sg1, se0, se1, ss0, ss1):
    c = lax.axis_index("c")
    s = lax.axis_index("s")
    sidx = (si0, si1, si2, si3)
    didx = (di0, di1, di2, di3)
    semsi = (ssi0, ssi1, ssi2, ssi3)
    semdi = (sdi0, sdi1, sdi2, sdi3)
    hrows = (hrows0, hrows1)
    eav = (ea0, ea1)
    pex = (pex0, pex1)
    semg = (sg0, sg1)
    seme = (se0, se1)
    sems = (ss0, ss1)

    pltpu.sync_copy(t_h, tvec_v)
    tv = tvec_v[...]

    start = s * NT_BASE + jnp.minimum(s, NT_REM)
    cnt = NT_BASE + (s < NT_REM).astype(jnp.int32)
    off = (c * N_NODES).astype(jnp.int32)

    @plsc.parallel_loop(0, MP_BLK, unroll=4)
    def _zrow(e):
        for j in range(8):
            pex0[e, pl.ds(j * 16, 16)] = jnp.zeros((16,), jnp.float32)

    nz = (ZCH - s + NS - 1) // NS
    def _zchunk(i, carry):
        k = s + i * NS
        pltpu.sync_copy(pex0, acc_sh.at[pl.ds(k * MP_BLK, MP_BLK), :])
        return carry
    lax.fori_loop(0, nz, _zchunk, 0)

    @pl.when(s == ZCH % NS)
    def _ztail():
        pltpu.sync_copy(pex0.at[pl.ds(0, ZTAIL), :],
                        acc_sh.at[pl.ds(ZCH * MP_BLK, ZTAIL), :])

    plsc.subcore_barrier()

    def _idx_issue(blk, q):
        pltpu.async_copy(src_h.at[pl.ds(start + blk, 1), :], sidx[q], semsi[q])
        pltpu.async_copy(dst_h.at[pl.ds(start + blk, 1), :], didx[q], semdi[q])

    def _idx_wait_shift(blk, q):
        pltpu.make_async_copy(src_h.at[pl.ds(start + blk, 1), :], sidx[q],
                              semsi[q]).wait()
        pltpu.make_async_copy(dst_h.at[pl.ds(start + blk, 1), :], didx[q],
                              semdi[q]).wait()
        for j in range(4):
            sidx[q][0, pl.ds(j * 16, 16)] = sidx[q][0, pl.ds(j * 16, 16)] + off

    def _in_issue(blk, q, b):
        pltpu.async_copy(hs_h.at[sidx[q].at[0]], hrows[b], semg[b])
        pltpu.async_copy(
            ea_h.at[pl.ds(c * N_EDGES + (start + blk) * MP_BLK, MP_BLK), :],
            eav[b], seme[b])

    def _in_wait(blk, q, b):
        pltpu.make_async_copy(hs_h.at[sidx[q].at[0]], hrows[b], semg[b]).wait()
        pltpu.make_async_copy(
            ea_h.at[pl.ds(c * N_EDGES + (start + blk) * MP_BLK, MP_BLK), :],
            eav[b], seme[b]).wait()

    def _compute(b):
        hr, av_, px = hrows[b], eav[b], pex[b]

        @plsc.parallel_loop(0, MP_BLK, unroll=4)
        def _erow(e):
            for j in range(4):
                hv = hr[e, pl.ds(j * 16, 16)]
                av = av_[e, pl.ds(j * 16, 16)]
                m = jnp.maximum(hv + av, 0.0) + 1e-7
                ex = jnp.exp(m * tv)
                px[e, pl.ds(j * 16, 16)] = m * ex
                px[e, pl.ds(64 + j * 16, 16)] = ex

    def _sc_start(q, b):
        pltpu.async_copy(pex[b], acc_sh.at[didx[q].at[0]], sems[b], add=True)

    def _sc_wait(q, b):
        pltpu.make_async_copy(pex[b], acc_sh.at[didx[q].at[0]], sems[b]).wait()

    _idx_issue(0, 0)
    _idx_issue(1, 1)
    _idx_wait_shift(0, 0)
    _in_issue(0, 0, 0)

    def _quad(it, carry):
        for u in range(4):
            i0 = 4 * it + u
            q = u
            b = u % 2
            qn = (u + 1) % 4
            bn = (u + 1) % 2

            @pl.when(i0 + 1 < cnt)
            def _nxt(i0=i0, qn=qn, bn=bn):
                _idx_wait_shift(i0 + 1, qn)
                _in_issue(i0 + 1, qn, bn)

            _in_wait(i0, q, b)

            @pl.when(i0 >= 2)
            def _wsc(q=q, b=b):
                _sc_wait((u + 2) % 4, b)

            _compute(b)
            _sc_start(q, b)

            @pl.when(i0 + 2 < cnt)
            def _iss(i0=i0, q=q):
                _idx_issue(i0 + 2, (u + 2) % 4)
        return carry

    lax.fori_loop(0, NT_BASE // 4, _quad, 0)

    @pl.when(s < NT_REM)
    def _tail():
        blk = NT_BASE
        _in_wait(blk, 0, 0)
        _sc_wait(2, 0)
        _compute(0)
        _sc_start(0, 0)
        _sc_wait(3, 1)
        _sc_wait(0, 0)

    @pl.when(s >= NT_REM)
    def _drain():
        _sc_wait(2, 0)
        _sc_wait(3, 1)

    plsc.subcore_barrier()

    def _wchunk(i, carry):
        k = s + i * NS
        pltpu.sync_copy(acc_sh.at[pl.ds(k * MP_BLK, MP_BLK), :],
                        out_h.at[pl.ds(c * N_NODES + k * MP_BLK, MP_BLK), :])
        return carry
    lax.fori_loop(0, nz, _wchunk, 0)

    @pl.when(s == ZCH % NS)
    def _wtail():
        pltpu.sync_copy(acc_sh.at[pl.ds(ZCH * MP_BLK, ZTAIL), :],
                        out_h.at[pl.ds(c * N_NODES + ZCH * MP_BLK, ZTAIL), :])


def _mp_sc(src, dst, hs2, ea2, t_arr):
    mesh = plsc.VectorSubcoreMesh(core_axis_name="c", subcore_axis_name="s",
                                  num_cores=NC, num_subcores=NS)
    idx_t = pltpu.VMEM((1, MP_BLK), jnp.int32)
    dma = pltpu.SemaphoreType.DMA
    f = pl.kernel(
        _mp_body,
        out_type=jax.ShapeDtypeStruct((NC * N_NODES, 128), jnp.float32),
        mesh=mesh,
        compiler_params=pltpu.CompilerParams(use_tc_tiling_on_sc=False),
        scratch_types=[
            pltpu.VMEM_SHARED((N_NODES, 128), jnp.float32),
            idx_t, idx_t, idx_t, idx_t,
            idx_t, idx_t, idx_t, idx_t,
            pltpu.VMEM((MP_BLK, HALF), jnp.float32),
            pltpu.VMEM((MP_BLK, HALF), jnp.float32),
            pltpu.VMEM((MP_BLK, HALF), jnp.float32),
            pltpu.VMEM((MP_BLK, HALF), jnp.float32),
            pltpu.VMEM((MP_BLK, 128), jnp.float32),
            pltpu.VMEM((MP_BLK, 128), jnp.float32),
            pltpu.VMEM((16,), jnp.float32),
            dma, dma, dma, dma,
            dma, dma, dma, dma,
            dma, dma, dma, dma, dma, dma,
        ],
    )
    return f(src.reshape(MP_NBLK, MP_BLK), dst.reshape(MP_NBLK, MP_BLK),
             hs2, ea2, t_arr)


def _mp(src, dst, hs_split, ea2, tl):
    hs2 = hs_split.reshape(NC * N_NODES, HALF)
    t_arr = jnp.full((16,), tl, jnp.float32)
    pex = _mp_sc(src, dst, hs2, ea2, t_arr)
    return pex.reshape(NC, N_NODES, 128)


def kernel(x, edge_index, edge_attr, W_ne, b_ne, W_ee, b_ee, t, W1, b1, g1,
           be1, W2, b2, ln_g, ln_b, W_out, b_out):
    src = edge_index[0]
    dst = edge_index[1]
    h_split = _encode_nodes(x, W_ne, b_ne)
    ea_split = _encode_edges(edge_attr, W_ee, b_ee)
    ea2 = ea_split.reshape(NC * N_EDGES, HALF)

    hs = h_split
    h = h_split
    for l in range(4):
        pex = _mp(src, dst, hs, ea2, t[l])
        nxt = (l + 1) % 4
        h, hs = _layer_post(pex, hs, h, W1[l], b1[l], g1[l], be1[l], W2[l],
                            b2[l], ln_g[nxt], ln_b[nxt], first=(l == 0))
    return _final(hs, W_out, b_out)


# speedup vs baseline: 9.0374x; 1.0001x over previous
import functools

import jax
import jax.numpy as jnp
from jax import lax
from jax.experimental import pallas as pl
from jax.experimental.pallas import tpu as pltpu
from jax.experimental.pallas import tpu_sc as plsc

N_NODES = 10000
N_EDGES = 320000
HID = 128
HALF = 64
N_CLASSES = 112

ROW_BLK = 1000
EDGE_BLK = 2000


def _ln(z, g, b):
    mu = jnp.mean(z, axis=-1, keepdims=True)
    var = jnp.mean((z - mu) ** 2, axis=-1, keepdims=True)
    return (z - mu) / jnp.sqrt(var + 1e-5) * g + b


def _enc_body(x_ref, w_ref, b_ref, out_ref):
    z = jnp.dot(x_ref[...], w_ref[...], preferred_element_type=jnp.float32)
    z = z + b_ref[...]
    out_ref[0, :, :] = z[:, :HALF]
    out_ref[1, :, :] = z[:, HALF:]


def _encode_nodes(x, W, b):
    return pl.pallas_call(
        _enc_body,
        grid=(N_NODES // ROW_BLK,),
        in_specs=[
            pl.BlockSpec((ROW_BLK, 128), lambda i: (i, 0)),
            pl.BlockSpec((128, 128), lambda i: (0, 0)),
            pl.BlockSpec((1, 128), lambda i: (0, 0)),
        ],
        out_specs=pl.BlockSpec((2, ROW_BLK, HALF), lambda i: (0, i, 0)),
        out_shape=jax.ShapeDtypeStruct((2, N_NODES, HALF), jnp.float32),
    )(x, W, b.reshape(1, 128))


def _eenc_body(x_ref, w_ref, b_ref, out_ref):
    z = jnp.dot(x_ref[...], w_ref[...], preferred_element_type=jnp.float32)
    z = z + b_ref[...]
    out_ref[0, :, :] = z[:, :HALF]
    out_ref[1, :, :] = z[:, HALF:]


def _encode_edges(ea, W, b):
    return pl.pallas_call(
        _eenc_body,
        grid=(N_EDGES // EDGE_BLK,),
        in_specs=[
            pl.BlockSpec((EDGE_BLK, 16), lambda i: (i, 0)),
            pl.BlockSpec((16, 128), lambda i: (0, 0)),
            pl.BlockSpec((1, 128), lambda i: (0, 0)),
        ],
        out_specs=pl.BlockSpec((2, EDGE_BLK, HALF), lambda i: (0, i, 0)),
        out_shape=jax.ShapeDtypeStruct((2, N_EDGES, HALF), jnp.float32),
    )(ea, W, b.reshape(1, 128))


def _post_body(pex_ref, hs_ref, hp_ref, w1_ref, b1_ref, g1_ref, be1_ref,
               w2_ref, b2_ref, lng_ref, lnb_ref, hnew_ref, hsn_ref, *, first):
    num = jnp.concatenate([pex_ref[0, :, :HALF], pex_ref[1, :, :HALF]], axis=1)
    den = jnp.concatenate([pex_ref[0, :, HALF:], pex_ref[1, :, HALF:]], axis=1)
    hs = jnp.concatenate([hs_ref[0], hs_ref[1]], axis=1)
    out = num / (den + 1e-16) + hs
    z = jnp.dot(out, w1_ref[...], preferred_element_type=jnp.float32) + b1_ref[...]
    z = _ln(z, g1_ref[...], be1_ref[...])
    z = jnp.maximum(z, 0.0)
    z = jnp.dot(z, w2_ref[...], preferred_element_type=jnp.float32) + b2_ref[...]
    if first:
        hnew = z
    else:
        hnew = jnp.concatenate([hp_ref[0], hp_ref[1]], axis=1) + z
    hsn = jnp.maximum(_ln(hnew, lng_ref[...], lnb_ref[...]), 0.0)
    hnew_ref[0, :, :] = hnew[:, :HALF]
    hnew_ref[1, :, :] = hnew[:, HALF:]
    hsn_ref[0, :, :] = hsn[:, :HALF]
    hsn_ref[1, :, :] = hsn[:, HALF:]


def _layer_post(pex, hs, hprev, W1l, b1l, g1l, be1l, W2l, b2l, lngn, lnbn, first):
    split = pl.BlockSpec((2, ROW_BLK, HALF), lambda i: (0, i, 0))
    row1 = lambda n: pl.BlockSpec((1, n), lambda i: (0, 0))
    return pl.pallas_call(
        functools.partial(_post_body, first=first),
        grid=(N_NODES // ROW_BLK,),
        in_specs=[
            pl.BlockSpec((2, ROW_BLK, 128), lambda i: (0, i, 0)),
            split,
            split,
            pl.BlockSpec((128, 256), lambda i: (0, 0)),
            row1(256), row1(256), row1(256),
            pl.BlockSpec((256, 128), lambda i: (0, 0)),
            row1(128), row1(128), row1(128),
        ],
        out_specs=[split, split],
        out_shape=[
            jax.ShapeDtypeStruct((2, N_NODES, HALF), jnp.float32),
            jax.ShapeDtypeStruct((2, N_NODES, HALF), jnp.float32),
        ],
    )(pex, hs, hprev, W1l, b1l.reshape(1, 256), g1l.reshape(1, 256),
      be1l.reshape(1, 256), W2l, b2l.reshape(1, 128), lngn.reshape(1, 128),
      lnbn.reshape(1, 128))


def _final_body(hs_ref, w_ref, b_ref, out_ref):
    hs = jnp.concatenate([hs_ref[0], hs_ref[1]], axis=1)
    z = jnp.dot(hs, w_ref[...], preferred_element_type=jnp.float32) + b_ref[...]
    m = jnp.max(z, axis=-1, keepdims=True)
    lse = jnp.log(jnp.sum(jnp.exp(z - m), axis=-1, keepdims=True)) + m
    out_ref[...] = z - lse


def _final(hs, W, b):
    return pl.pallas_call(
        _final_body,
        grid=(N_NODES // ROW_BLK,),
        in_specs=[
            pl.BlockSpec((2, ROW_BLK, HALF), lambda i: (0, i, 0)),
            pl.BlockSpec((128, N_CLASSES), lambda i: (0, 0)),
            pl.BlockSpec((1, N_CLASSES), lambda i: (0, 0)),
        ],
        out_specs=pl.BlockSpec((ROW_BLK, N_CLASSES), lambda i: (i, 0)),
        out_shape=jax.ShapeDtypeStruct((N_NODES, N_CLASSES), jnp.float32),
    )(hs, W, b.reshape(1, N_CLASSES))


MP_BLK = 64
MP_NBLK = N_EDGES // MP_BLK
NS = 16
NC = 2
ZCH = N_NODES // MP_BLK
ZTAIL = N_NODES - ZCH * MP_BLK
NT_BASE = MP_NBLK // NS
NT_REM = MP_NBLK - NT_BASE * NS


def _mp_body(src_h, dst_h, hs_h, ea_h, t_h, out_h,
             acc_sh,
             si0, si1, si2, si3, di0, di1, di2, di3,
             hrows0, hrows1, ea0, ea1, pex0, pex1, tvec_v,
             ssi0, ssi1, ssi2, ssi3, sdi0, sdi1, sdi2, sdi3,
             sg0, sg1, se0, se1, ss0, ss1):
    c = lax.axis_index("c")
    s = lax.axis_index("s")
    sidx = (si0, si1, si2, si3)
    didx = (di0, di1, di2, di3)
    semsi = (ssi0, ssi1, ssi2, ssi3)
    semdi = (sdi0, sdi1, sdi2, sdi3)
    hrows = (hrows0, hrows1)
    eav = (ea0, ea1)
    pex = (pex0, pex1)
    semg = (sg0, sg1)
    seme = (se0, se1)
    sems = (ss0, ss1)

    pltpu.sync_copy(t_h, tvec_v)
    tv = tvec_v[...]

    start = s * NT_BASE + jnp.minimum(s, NT_REM)
    cnt = NT_BASE + (s < NT_REM).astype(jnp.int32)
    off = (c * N_NODES).astype(jnp.int32)

    @plsc.parallel_loop(0, MP_BLK, unroll=4)
    def _zrow(e):
        for j in range(8):
            pex0[e, pl.ds(j * 16, 16)] = jnp.zeros((16,), jnp.float32)

    nz = (ZCH - s + NS - 1) // NS
    def _zchunk(i, carry):
        k = s + i * NS
        pltpu.sync_copy(pex0, acc_sh.at[pl.ds(k * MP_BLK, MP_BLK), :])
        return carry
    lax.fori_loop(0, nz, _zchunk, 0)

    @pl.when(s == ZCH % NS)
    def _ztail():
        pltpu.sync_copy(pex0.at[pl.ds(0, ZTAIL), :],
                        acc_sh.at[pl.ds(ZCH * MP_BLK, ZTAIL), :])

    plsc.subcore_barrier()

    def _idx_issue(blk, q):
        pltpu.async_copy(src_h.at[pl.ds(start + blk, 1), :], sidx[q], semsi[q])
        pltpu.async_copy(dst_h.at[pl.ds(start + blk, 1), :], didx[q], semdi[q])

    def _idx_wait_shift(blk, q):
        pltpu.make_async_copy(src_h.at[pl.ds(start + blk, 1), :], sidx[q],
                              semsi[q]).wait()
        pltpu.make_async_copy(dst_h.at[pl.ds(start + blk, 1), :], didx[q],
                              semdi[q]).wait()
        for j in range(4):
            sidx[q][0, pl.ds(j * 16, 16)] = sidx[q][0, pl.ds(j * 16, 16)] + off

    def _in_issue(blk, q, b):
        pltpu.async_copy(hs_h.at[sidx[q].at[0]], hrows[b], semg[b])
        pltpu.async_copy(
            ea_h.at[pl.ds(c * N_EDGES + (start + blk) * MP_BLK, MP_BLK), :],
            eav[b], seme[b])

    def _in_wait(blk, q, b):
        pltpu.make_async_copy(hs_h.at[sidx[q].at[0]], hrows[b], semg[b]).wait()
        pltpu.make_async_copy(
            ea_h.at[pl.ds(c * N_EDGES + (start + blk) * MP_BLK, MP_BLK), :],
            eav[b], seme[b]).wait()

    def _compute(b):
        hr, av_, px = hrows[b], eav[b], pex[b]

        @plsc.parallel_loop(0, MP_BLK, unroll=4)
        def _erow(e):
            for j in range(4):
                hv = hr[e, pl.ds(j * 16, 16)]
                av = av_[e, pl.ds(j * 16, 16)]
                m = jnp.maximum(hv + av, 0.0) + 1e-7
                ex = jnp.exp(m * tv)
                px[e, pl.ds(j * 16, 16)] = m * ex
                px[e, pl.ds(64 + j * 16, 16)] = ex

    def _sc_start(q, b):
        pltpu.async_copy(pex[b], acc_sh.at[didx[q].at[0]], sems[b], add=True)

    def _sc_wait(q, b):
        pltpu.make_async_copy(pex[b], acc_sh.at[didx[q].at[0]], sems[b]).wait()

    _idx_issue(0, 0)
    _idx_issue(1, 1)
    _idx_wait_shift(0, 0)
    _in_issue(0, 0, 0)

    def _quad(it, carry):
        for u in range(4):
            i0 = 4 * it + u
            q = u
            b = u % 2
            qn = (u + 1) % 4
            bn = (u + 1) % 2

            @pl.when(i0 + 1 < cnt)
            def _nxt(i0=i0, qn=qn, bn=bn):
                _idx_wait_shift(i0 + 1, qn)
                _in_issue(i0 + 1, qn, bn)

            _in_wait(i0, q, b)

            @pl.when(i0 >= 2)
            def _wsc(q=q, b=b):
                _sc_wait((u + 2) % 4, b)

            _compute(b)
            _sc_start(q, b)

            @pl.when(i0 + 2 < cnt)
            def _iss(i0=i0, q=q):
                _idx_issue(i0 + 2, (u + 2) % 4)
        return carry

    lax.fori_loop(0, NT_BASE // 4, _quad, 0)

    @pl.when(s < NT_REM)
    def _tail():
        blk = NT_BASE
        _in_wait(blk, 0, 0)
        _sc_wait(2, 0)
        _compute(0)
        _sc_start(0, 0)
        _sc_wait(3, 1)
        _sc_wait(0, 0)

    @pl.when(s >= NT_REM)
    def _drain():
        _sc_wait(2, 0)
        _sc_wait(3, 1)

    plsc.subcore_barrier()

    def _wchunk(i, carry):
        k = s + i * NS
        pltpu.sync_copy(acc_sh.at[pl.ds(k * MP_BLK, MP_BLK), :],
                        out_h.at[pl.ds(c * N_NODES + k * MP_BLK, MP_BLK), :])
        return carry
    lax.fori_loop(0, nz, _wchunk, 0)

    @pl.when(s == ZCH % NS)
    def _wtail():
        pltpu.sync_copy(acc_sh.at[pl.ds(ZCH * MP_BLK, ZTAIL), :],
                        out_h.at[pl.ds(c * N_NODES + ZCH * MP_BLK, ZTAIL), :])


def _mp_sc(src, dst, hs2, ea2, t_arr):
    mesh = plsc.VectorSubcoreMesh(core_axis_name="c", subcore_axis_name="s",
                                  num_cores=NC, num_subcores=NS)
    idx_t = pltpu.VMEM((1, MP_BLK), jnp.int32)
    dma = pltpu.SemaphoreType.DMA
    f = pl.kernel(
        _mp_body,
        out_type=jax.ShapeDtypeStruct((NC * N_NODES, 128), jnp.float32),
        mesh=mesh,
        compiler_params=pltpu.CompilerParams(use_tc_tiling_on_sc=False, skip_device_barrier=True),
        scratch_types=[
            pltpu.VMEM_SHARED((N_NODES, 128), jnp.float32),
            idx_t, idx_t, idx_t, idx_t,
            idx_t, idx_t, idx_t, idx_t,
            pltpu.VMEM((MP_BLK, HALF), jnp.float32),
            pltpu.VMEM((MP_BLK, HALF), jnp.float32),
            pltpu.VMEM((MP_BLK, HALF), jnp.float32),
            pltpu.VMEM((MP_BLK, HALF), jnp.float32),
            pltpu.VMEM((MP_BLK, 128), jnp.float32),
            pltpu.VMEM((MP_BLK, 128), jnp.float32),
            pltpu.VMEM((16,), jnp.float32),
            dma, dma, dma, dma,
            dma, dma, dma, dma,
            dma, dma, dma, dma, dma, dma,
        ],
    )
    return f(src.reshape(MP_NBLK, MP_BLK), dst.reshape(MP_NBLK, MP_BLK),
             hs2, ea2, t_arr)


def _mp(src, dst, hs_split, ea2, tl):
    hs2 = hs_split.reshape(NC * N_NODES, HALF)
    t_arr = jnp.full((16,), tl, jnp.float32)
    pex = _mp_sc(src, dst, hs2, ea2, t_arr)
    return pex.reshape(NC, N_NODES, 128)


def kernel(x, edge_index, edge_attr, W_ne, b_ne, W_ee, b_ee, t, W1, b1, g1,
           be1, W2, b2, ln_g, ln_b, W_out, b_out):
    src = edge_index[0]
    dst = edge_index[1]
    h_split = _encode_nodes(x, W_ne, b_ne)
    ea_split = _encode_edges(edge_attr, W_ee, b_ee)
    ea2 = ea_split.reshape(NC * N_EDGES, HALF)

    hs = h_split
    h = h_split
    for l in range(4):
        pex = _mp(src, dst, hs, ea2, t[l])
        nxt = (l + 1) % 4
        h, hs = _layer_post(pex, hs, h, W1[l], b1[l], g1[l], be1[l], W2[l],
                            b2[l], ln_g[nxt], ln_b[nxt], first=(l == 0))
    return _final(hs, W_out, b_out)


# speedup vs baseline: 9.0484x; 1.0012x over previous
import functools

import jax
import jax.numpy as jnp
from jax import lax
from jax.experimental import pallas as pl
from jax.experimental.pallas import tpu as pltpu
from jax.experimental.pallas import tpu_sc as plsc

N_NODES = 10000
N_EDGES = 320000
HID = 128
HALF = 64
N_CLASSES = 112

ROW_BLK = 1000
EDGE_BLK = 2000


def _ln(z, g, b):
    mu = jnp.mean(z, axis=-1, keepdims=True)
    var = jnp.mean((z - mu) ** 2, axis=-1, keepdims=True)
    return (z - mu) / jnp.sqrt(var + 1e-5) * g + b


def _enc_body(x_ref, w_ref, b_ref, out_ref):
    z = jnp.dot(x_ref[...], w_ref[...], preferred_element_type=jnp.float32)
    z = z + b_ref[...]
    out_ref[0, :, :] = z[:, :HALF]
    out_ref[1, :, :] = z[:, HALF:]


def _encode_nodes(x, W, b):
    return pl.pallas_call(
        _enc_body,
        grid=(N_NODES // ROW_BLK,),
        in_specs=[
            pl.BlockSpec((ROW_BLK, 128), lambda i: (i, 0)),
            pl.BlockSpec((128, 128), lambda i: (0, 0)),
            pl.BlockSpec((1, 128), lambda i: (0, 0)),
        ],
        out_specs=pl.BlockSpec((2, ROW_BLK, HALF), lambda i: (0, i, 0)),
        out_shape=jax.ShapeDtypeStruct((2, N_NODES, HALF), jnp.float32),
    )(x, W, b.reshape(1, 128))


def _eenc_body(x_ref, w_ref, b_ref, out_ref):
    z = jnp.dot(x_ref[...], w_ref[...], preferred_element_type=jnp.float32)
    z = z + b_ref[...]
    out_ref[0, :, :] = z[:, :HALF]
    out_ref[1, :, :] = z[:, HALF:]


def _encode_edges(ea, W, b):
    return pl.pallas_call(
        _eenc_body,
        grid=(N_EDGES // EDGE_BLK,),
        in_specs=[
            pl.BlockSpec((EDGE_BLK, 16), lambda i: (i, 0)),
            pl.BlockSpec((16, 128), lambda i: (0, 0)),
            pl.BlockSpec((1, 128), lambda i: (0, 0)),
        ],
        out_specs=pl.BlockSpec((2, EDGE_BLK, HALF), lambda i: (0, i, 0)),
        out_shape=jax.ShapeDtypeStruct((2, N_EDGES, HALF), jnp.float32),
    )(ea, W, b.reshape(1, 128))


def _post_body(pex_ref, hs_ref, hp_ref, w1_ref, b1_ref, g1_ref, be1_ref,
               w2_ref, b2_ref, lng_ref, lnb_ref, hnew_ref, hsn_ref, *, first):
    num = jnp.concatenate([pex_ref[0, :, :HALF], pex_ref[1, :, :HALF]], axis=1)
    den = jnp.concatenate([pex_ref[0, :, HALF:], pex_ref[1, :, HALF:]], axis=1)
    hs = jnp.concatenate([hs_ref[0], hs_ref[1]], axis=1)
    out = num / (den + 1e-16) + hs
    z = jnp.dot(out, w1_ref[...], preferred_element_type=jnp.float32) + b1_ref[...]
    z = _ln(z, g1_ref[...], be1_ref[...])
    z = jnp.maximum(z, 0.0)
    z = jnp.dot(z, w2_ref[...], preferred_element_type=jnp.float32) + b2_ref[...]
    if first:
        hnew = z
    else:
        hnew = jnp.concatenate([hp_ref[0], hp_ref[1]], axis=1) + z
    hsn = jnp.maximum(_ln(hnew, lng_ref[...], lnb_ref[...]), 0.0)
    hnew_ref[0, :, :] = hnew[:, :HALF]
    hnew_ref[1, :, :] = hnew[:, HALF:]
    hsn_ref[0, :, :] = hsn[:, :HALF]
    hsn_ref[1, :, :] = hsn[:, HALF:]


def _layer_post(pex, hs, hprev, W1l, b1l, g1l, be1l, W2l, b2l, lngn, lnbn, first):
    split = pl.BlockSpec((2, ROW_BLK, HALF), lambda i: (0, i, 0))
    row1 = lambda n: pl.BlockSpec((1, n), lambda i: (0, 0))
    return pl.pallas_call(
        functools.partial(_post_body, first=first),
        grid=(N_NODES // ROW_BLK,),
        in_specs=[
            pl.BlockSpec((2, ROW_BLK, 128), lambda i: (0, i, 0)),
            split,
            split,
            pl.BlockSpec((128, 256), lambda i: (0, 0)),
            row1(256), row1(256), row1(256),
            pl.BlockSpec((256, 128), lambda i: (0, 0)),
            row1(128), row1(128), row1(128),
        ],
        out_specs=[split, split],
        out_shape=[
            jax.ShapeDtypeStruct((2, N_NODES, HALF), jnp.float32),
            jax.ShapeDtypeStruct((2, N_NODES, HALF), jnp.float32),
        ],
    )(pex, hs, hprev, W1l, b1l.reshape(1, 256), g1l.reshape(1, 256),
      be1l.reshape(1, 256), W2l, b2l.reshape(1, 128), lngn.reshape(1, 128),
      lnbn.reshape(1, 128))


def _final_body(hs_ref, w_ref, b_ref, out_ref):
    hs = jnp.concatenate([hs_ref[0], hs_ref[1]], axis=1)
    z = jnp.dot(hs, w_ref[...], preferred_element_type=jnp.float32) + b_ref[...]
    m = jnp.max(z, axis=-1, keepdims=True)
    lse = jnp.log(jnp.sum(jnp.exp(z - m), axis=-1, keepdims=True)) + m
    out_ref[...] = z - lse


def _final(hs, W, b):
    return pl.pallas_call(
        _final_body,
        grid=(N_NODES // ROW_BLK,),
        in_specs=[
            pl.BlockSpec((2, ROW_BLK, HALF), lambda i: (0, i, 0)),
            pl.BlockSpec((128, N_CLASSES), lambda i: (0, 0)),
            pl.BlockSpec((1, N_CLASSES), lambda i: (0, 0)),
        ],
        out_specs=pl.BlockSpec((ROW_BLK, N_CLASSES), lambda i: (i, 0)),
        out_shape=jax.ShapeDtypeStruct((N_NODES, N_CLASSES), jnp.float32),
    )(hs, W, b.reshape(1, N_CLASSES))


MP_BLK = 64
MP_NBLK = N_EDGES // MP_BLK
NS = 16
NC = 2
ZCH = N_NODES // MP_BLK
ZTAIL = N_NODES - ZCH * MP_BLK
NT_BASE = MP_NBLK // NS
NT_REM = MP_NBLK - NT_BASE * NS


def _mp_body(src_h, dst_h, hs_h, ea_h, t_h, out_h,
             acc_sh,
             si0, si1, si2, si3, di0, di1, di2, di3,
             hrows0, hrows1, ea0, ea1, pex0, pex1, tvec_v,
             ssi0, ssi1, ssi2, ssi3, sdi0, sdi1, sdi2, sdi3,
             sg0, sg1, se0, se1, ss0, ss1):
    c = lax.axis_index("c")
    s = lax.axis_index("s")
    sidx = (si0, si1, si2, si3)
    didx = (di0, di1, di2, di3)
    semsi = (ssi0, ssi1, ssi2, ssi3)
    semdi = (sdi0, sdi1, sdi2, sdi3)
    hrows = (hrows0, hrows1)
    eav = (ea0, ea1)
    pex = (pex0, pex1)
    semg = (sg0, sg1)
    seme = (se0, se1)
    sems = (ss0, ss1)

    pltpu.sync_copy(t_h, tvec_v)
    tv = tvec_v[...]

    start = s * NT_BASE + jnp.minimum(s, NT_REM)
    cnt = NT_BASE + (s < NT_REM).astype(jnp.int32)
    off = (c * N_NODES).astype(jnp.int32)

    @plsc.parallel_loop(0, MP_BLK, unroll=4)
    def _zrow(e):
        for j in range(8):
            pex0[e, pl.ds(j * 16, 16)] = jnp.zeros((16,), jnp.float32)

    nz = (ZCH - s + NS - 1) // NS
    def _zchunk(i, carry):
        k = s + i * NS
        pltpu.sync_copy(pex0, acc_sh.at[pl.ds(k * MP_BLK, MP_BLK), :])
        return carry
    lax.fori_loop(0, nz, _zchunk, 0)

    @pl.when(s == ZCH % NS)
    def _ztail():
        pltpu.sync_copy(pex0.at[pl.ds(0, ZTAIL), :],
                        acc_sh.at[pl.ds(ZCH * MP_BLK, ZTAIL), :])

    plsc.subcore_barrier()

    def _idx_issue(blk, q):
        pltpu.async_copy(src_h.at[pl.ds(start + blk, 1), :], sidx[q], semsi[q])
        pltpu.async_copy(dst_h.at[pl.ds(start + blk, 1), :], didx[q], semdi[q])

    def _idx_wait_shift(blk, q):
        pltpu.make_async_copy(src_h.at[pl.ds(start + blk, 1), :], sidx[q],
                              semsi[q]).wait()
        pltpu.make_async_copy(dst_h.at[pl.ds(start + blk, 1), :], didx[q],
                              semdi[q]).wait()
        for j in range(4):
            sidx[q][0, pl.ds(j * 16, 16)] = sidx[q][0, pl.ds(j * 16, 16)] + off

    def _in_issue(blk, q, b):
        pltpu.async_copy(hs_h.at[sidx[q].at[0]], hrows[b], semg[b])
        pltpu.async_copy(
            ea_h.at[c, pl.ds((start + blk) * MP_BLK, MP_BLK), :],
            eav[b], seme[b])

    def _in_wait(blk, q, b):
        pltpu.make_async_copy(hs_h.at[sidx[q].at[0]], hrows[b], semg[b]).wait()
        pltpu.make_async_copy(
            ea_h.at[c, pl.ds((start + blk) * MP_BLK, MP_BLK), :],
            eav[b], seme[b]).wait()

    def _compute(b):
        hr, av_, px = hrows[b], eav[b], pex[b]

        @plsc.parallel_loop(0, MP_BLK, unroll=4)
        def _erow(e):
            for j in range(4):
                hv = hr[e, pl.ds(j * 16, 16)]
                av = av_[e, pl.ds(j * 16, 16)]
                m = jnp.maximum(hv + av, 0.0) + 1e-7
                ex = jnp.exp(m * tv)
                px[e, pl.ds(j * 16, 16)] = m * ex
                px[e, pl.ds(64 + j * 16, 16)] = ex

    def _sc_start(q, b):
        pltpu.async_copy(pex[b], acc_sh.at[didx[q].at[0]], sems[b], add=True)

    def _sc_wait(q, b):
        pltpu.make_async_copy(pex[b], acc_sh.at[didx[q].at[0]], sems[b]).wait()

    _idx_issue(0, 0)
    _idx_issue(1, 1)
    _idx_wait_shift(0, 0)
    _in_issue(0, 0, 0)

    def _quad(it, carry):
        for u in range(4):
            i0 = 4 * it + u
            q = u
            b = u % 2
            qn = (u + 1) % 4
            bn = (u + 1) % 2

            @pl.when(i0 + 1 < cnt)
            def _nxt(i0=i0, qn=qn, bn=bn):
                _idx_wait_shift(i0 + 1, qn)
                _in_issue(i0 + 1, qn, bn)

            _in_wait(i0, q, b)

            @pl.when(i0 >= 2)
            def _wsc(q=q, b=b):
                _sc_wait((u + 2) % 4, b)

            _compute(b)
            _sc_start(q, b)

            @pl.when(i0 + 2 < cnt)
            def _iss(i0=i0, q=q):
                _idx_issue(i0 + 2, (u + 2) % 4)
        return carry

    lax.fori_loop(0, NT_BASE // 4, _quad, 0)

    @pl.when(s < NT_REM)
    def _tail():
        blk = NT_BASE
        _in_wait(blk, 0, 0)
        _sc_wait(2, 0)
        _compute(0)
        _sc_start(0, 0)
        _sc_wait(3, 1)
        _sc_wait(0, 0)

    @pl.when(s >= NT_REM)
    def _drain():
        _sc_wait(2, 0)
        _sc_wait(3, 1)

    plsc.subcore_barrier()

    def _wchunk(i, carry):
        k = s + i * NS
        pltpu.sync_copy(acc_sh.at[pl.ds(k * MP_BLK, MP_BLK), :],
                        out_h.at[c, pl.ds(k * MP_BLK, MP_BLK), :])
        return carry
    lax.fori_loop(0, nz, _wchunk, 0)

    @pl.when(s == ZCH % NS)
    def _wtail():
        pltpu.sync_copy(acc_sh.at[pl.ds(ZCH * MP_BLK, ZTAIL), :],
                        out_h.at[c, pl.ds(ZCH * MP_BLK, ZTAIL), :])


def _mp_sc(src, dst, hs2, ea2, t_arr):
    mesh = plsc.VectorSubcoreMesh(core_axis_name="c", subcore_axis_name="s",
                                  num_cores=NC, num_subcores=NS)
    idx_t = pltpu.VMEM((1, MP_BLK), jnp.int32)
    dma = pltpu.SemaphoreType.DMA
    f = pl.kernel(
        _mp_body,
        out_type=jax.ShapeDtypeStruct((NC, N_NODES, 128), jnp.float32),
        mesh=mesh,
        compiler_params=pltpu.CompilerParams(use_tc_tiling_on_sc=False, skip_device_barrier=True),
        scratch_types=[
            pltpu.VMEM_SHARED((N_NODES, 128), jnp.float32),
            idx_t, idx_t, idx_t, idx_t,
            idx_t, idx_t, idx_t, idx_t,
            pltpu.VMEM((MP_BLK, HALF), jnp.float32),
            pltpu.VMEM((MP_BLK, HALF), jnp.float32),
            pltpu.VMEM((MP_BLK, HALF), jnp.float32),
            pltpu.VMEM((MP_BLK, HALF), jnp.float32),
            pltpu.VMEM((MP_BLK, 128), jnp.float32),
            pltpu.VMEM((MP_BLK, 128), jnp.float32),
            pltpu.VMEM((16,), jnp.float32),
            dma, dma, dma, dma,
            dma, dma, dma, dma,
            dma, dma, dma, dma, dma, dma,
        ],
    )
    return f(src.reshape(MP_NBLK, MP_BLK), dst.reshape(MP_NBLK, MP_BLK),
             hs2, ea2, t_arr)


def _mp(src, dst, hs_split, ea3, tl):
    hs2 = hs_split.reshape(NC * N_NODES, HALF)
    t_arr = jnp.full((16,), tl, jnp.float32)
    return _mp_sc(src, dst, hs2, ea3, t_arr)


def kernel(x, edge_index, edge_attr, W_ne, b_ne, W_ee, b_ee, t, W1, b1, g1,
           be1, W2, b2, ln_g, ln_b, W_out, b_out):
    src = edge_index[0]
    dst = edge_index[1]
    h_split = _encode_nodes(x, W_ne, b_ne)
    ea_split = _encode_edges(edge_attr, W_ee, b_ee)

    hs = h_split
    h = h_split
    for l in range(4):
        pex = _mp(src, dst, hs, ea_split, t[l])
        nxt = (l + 1) % 4
        h, hs = _layer_post(pex, hs, h, W1[l], b1[l], g1[l], be1[l], W2[l],
                            b2[l], ln_g[nxt], ln_b[nxt], first=(l == 0))
    return _final(hs, W_out, b_out)
